# baseline (device time: 38078 ns/iter reference)
import jax
import jax.numpy as jnp
from jax import lax
from jax.experimental import pallas as pl
from jax.experimental.pallas import tpu as pltpu


def kernel(partial, resid, gamma):
    M, D = resid.shape
    half = M // 2
    p2 = partial.reshape(M, D)
    g2 = gamma.reshape(1, D)

    n_chunks = 16
    rows = half // n_chunks

    def body(p_ref, r_ref, g_ref, o_ref, pv, yrecv, rv, ov, gv,
             copy_sems, ysend_sems, yrecv_sems, xsend_sems, xrecv_sems,
             osave_sems):
        my_x = lax.axis_index("x")
        my_y = lax.axis_index("y")
        row0 = my_x * half
        y_nbr = (my_x, 1 - my_y)
        x_nbr = (1 - my_x, my_y)

        cp_p = pltpu.make_async_copy(
            p_ref.at[pl.ds(row0, half), :], pv, copy_sems.at[0])
        cp_r = pltpu.make_async_copy(
            r_ref.at[pl.ds(row0, half), :], rv, copy_sems.at[1])
        cp_g = pltpu.make_async_copy(g_ref, gv, copy_sems.at[2])
        cp_p.start()
        cp_r.start()
        cp_g.start()

        barrier_sem = pltpu.get_barrier_semaphore()
        for nbr in (y_nbr, x_nbr):
            pl.semaphore_signal(
                barrier_sem, inc=1,
                device_id=nbr, device_id_type=pl.DeviceIdType.MESH,
            )
        pl.semaphore_wait(barrier_sem, 2)

        rdmas_y = []
        for c in range(n_chunks):
            rdma_y = pltpu.make_async_remote_copy(
                src_ref=p_ref.at[pl.ds(row0 + c * rows, rows), :],
                dst_ref=yrecv.at[pl.ds(c * rows, rows), :],
                send_sem=ysend_sems.at[c],
                recv_sem=yrecv_sems.at[c],
                device_id=y_nbr,
                device_id_type=pl.DeviceIdType.MESH,
            )
            rdma_y.start()
            rdmas_y.append(rdma_y)

        cp_p.wait()
        cp_r.wait()
        cp_g.wait()

        rdmas_x = []
        saves = []
        for c in range(n_chunks):
            rdmas_y[c].wait_recv()
            sl = pl.ds(c * rows, rows)
            osl = pl.ds(row0 + c * rows, rows)
            yh = pv[sl, :] + yrecv[sl, :] + rv[sl, :]
            rms = jnp.sqrt(jnp.mean(yh * yh, axis=1, keepdims=True) + 1e-6)
            ov[sl, :] = yh / rms * gv[:, :]

            rdma_x = pltpu.make_async_remote_copy(
                src_ref=ov.at[sl, :],
                dst_ref=o_ref.at[osl, :],
                send_sem=xsend_sems.at[c],
                recv_sem=xrecv_sems.at[c],
                device_id=x_nbr,
                device_id_type=pl.DeviceIdType.MESH,
            )
            rdma_x.start()
            rdmas_x.append(rdma_x)

            save = pltpu.make_async_copy(
                ov.at[sl, :], o_ref.at[osl, :], osave_sems.at[c])
            save.start()
            saves.append(save)

        for c in range(n_chunks):
            rdmas_y[c].wait_send()
            rdmas_x[c].wait()
            saves[c].wait()

    return pl.pallas_call(
        body,
        out_shape=jax.ShapeDtypeStruct((M, D), jnp.float32),
        in_specs=[
            pl.BlockSpec(memory_space=pl.ANY),
            pl.BlockSpec(memory_space=pl.ANY),
            pl.BlockSpec(memory_space=pl.ANY),
        ],
        out_specs=pl.BlockSpec(memory_space=pl.ANY),
        scratch_shapes=[
            pltpu.VMEM((half, D), jnp.float32),
            pltpu.VMEM((half, D), jnp.float32),
            pltpu.VMEM((half, D), jnp.float32),
            pltpu.VMEM((half, D), jnp.float32),
            pltpu.VMEM((1, D), jnp.float32),
            pltpu.SemaphoreType.DMA((3,)),
            pltpu.SemaphoreType.DMA((n_chunks,)),
            pltpu.SemaphoreType.DMA((n_chunks,)),
            pltpu.SemaphoreType.DMA((n_chunks,)),
            pltpu.SemaphoreType.DMA((n_chunks,)),
            pltpu.SemaphoreType.DMA((n_chunks,)),
        ],
        compiler_params=pltpu.CompilerParams(collective_id=0),
    )(p2, resid, g2)


# device time: 38060 ns/iter; 1.0005x vs baseline; 1.0005x over previous
import jax
import jax.numpy as jnp
from jax import lax
from jax.experimental import pallas as pl
from jax.experimental.pallas import tpu as pltpu


def kernel(partial, resid, gamma):
    M, D = resid.shape
    half = M // 2
    p2 = partial.reshape(M, D)
    g2 = gamma.reshape(1, D)

    n_chunks = 16
    rows = half // n_chunks

    def body(p_ref, r_ref, g_ref, o_ref, pv, yrecv, rv, ov, gv,
             copy_sems, ysend_sems, yrecv_sems, xsend_sems, xrecv_sems,
             osave_sems):
        my_x = lax.axis_index("x")
        my_y = lax.axis_index("y")
        row0 = my_x * half
        y_nbr = (my_x, 1 - my_y)
        x_nbr = (1 - my_x, my_y)

        cp_p = pltpu.make_async_copy(
            p_ref.at[pl.ds(row0, half), :], pv, copy_sems.at[0])
        cp_r = pltpu.make_async_copy(
            r_ref.at[pl.ds(row0, half), :], rv, copy_sems.at[1])
        cp_g = pltpu.make_async_copy(g_ref, gv, copy_sems.at[2])
        cp_p.start()
        cp_r.start()
        cp_g.start()

        barrier_sem = pltpu.get_barrier_semaphore()
        for nbr in (y_nbr, x_nbr):
            pl.semaphore_signal(
                barrier_sem, inc=1,
                device_id=nbr, device_id_type=pl.DeviceIdType.MESH,
            )
        pl.semaphore_wait(barrier_sem, 2)

        rdmas_y = []
        for c in range(n_chunks):
            rdma_y = pltpu.make_async_remote_copy(
                src_ref=p_ref.at[pl.ds(row0 + c * rows, rows), :],
                dst_ref=yrecv.at[pl.ds(c * rows, rows), :],
                send_sem=ysend_sems.at[c],
                recv_sem=yrecv_sems.at[c],
                device_id=y_nbr,
                device_id_type=pl.DeviceIdType.MESH,
            )
            rdma_y.start()
            rdmas_y.append(rdma_y)

        cp_p.wait()
        cp_r.wait()
        cp_g.wait()

        rdmas_x = []
        saves = []
        for c in range(n_chunks):
            rdmas_y[c].wait_recv()
            sl = pl.ds(c * rows, rows)
            osl = pl.ds(row0 + c * rows, rows)
            yh = pv[sl, :] + yrecv[sl, :] + rv[sl, :]
            rms = jnp.sqrt(jnp.mean(yh * yh, axis=1, keepdims=True) + 1e-6)
            ov[sl, :] = yh / rms * gv[:, :]

            rdma_x = pltpu.make_async_remote_copy(
                src_ref=ov.at[sl, :],
                dst_ref=o_ref.at[osl, :],
                send_sem=xsend_sems.at[c],
                recv_sem=xrecv_sems.at[c],
                device_id=x_nbr,
                device_id_type=pl.DeviceIdType.MESH,
            )
            rdma_x.start()
            rdmas_x.append(rdma_x)

            save = pltpu.make_async_copy(
                ov.at[sl, :], o_ref.at[osl, :], osave_sems.at[c])
            save.start()
            saves.append(save)

        for c in range(n_chunks):
            rdmas_y[c].wait_send()
            rdmas_x[c].wait()
            saves[c].wait()

    return pl.pallas_call(
        body,
        out_shape=jax.ShapeDtypeStruct((M, D), jnp.float32),
        in_specs=[
            pl.BlockSpec(memory_space=pltpu.MemorySpace.HBM),
            pl.BlockSpec(memory_space=pltpu.MemorySpace.HBM),
            pl.BlockSpec(memory_space=pltpu.MemorySpace.HBM),
        ],
        out_specs=pl.BlockSpec(memory_space=pltpu.MemorySpace.HBM),
        scratch_shapes=[
            pltpu.VMEM((half, D), jnp.float32),
            pltpu.VMEM((half, D), jnp.float32),
            pltpu.VMEM((half, D), jnp.float32),
            pltpu.VMEM((half, D), jnp.float32),
            pltpu.VMEM((1, D), jnp.float32),
            pltpu.SemaphoreType.DMA((3,)),
            pltpu.SemaphoreType.DMA((n_chunks,)),
            pltpu.SemaphoreType.DMA((n_chunks,)),
            pltpu.SemaphoreType.DMA((n_chunks,)),
            pltpu.SemaphoreType.DMA((n_chunks,)),
            pltpu.SemaphoreType.DMA((n_chunks,)),
        ],
        compiler_params=pltpu.CompilerParams(collective_id=0),
    )(p2, resid, g2)
